# baseline (speedup 1.0000x reference)
import jax
import jax.numpy as jnp
from jax.experimental import pallas as pl
from jax.experimental.pallas import tpu as pltpu

B, S, D = 4, 2048, 1024
ROWS = B * S
CHUNK_ROWS = 256
NCHUNK = ROWS // CHUNK_ROWS
PE_CH = S // CHUNK_ROWS
DEPTH = 8


def _body(x_hbm, pe_hbm, out_hbm, pe_v, x_v, o_v, pe_sem, x_sem, o_sem):
    def pe_copy(p):
        return pltpu.make_async_copy(
            pe_hbm.at[pl.ds(p * CHUNK_ROWS, CHUNK_ROWS), :],
            pe_v.at[p], pe_sem.at[p])

    def x_copy(c, slot):
        return pltpu.make_async_copy(
            x_hbm.at[pl.ds(c * CHUNK_ROWS, CHUNK_ROWS), :],
            x_v.at[slot], x_sem.at[slot])

    def o_copy(c, slot):
        return pltpu.make_async_copy(
            o_v.at[slot],
            out_hbm.at[pl.ds(c * CHUNK_ROWS, CHUNK_ROWS), :],
            o_sem.at[slot])

    for c in range(DEPTH):
        x_copy(c, c).start()
        pe_copy(c).start()

    for c in range(NCHUNK):
        slot = c % DEPTH
        p = c % PE_CH
        x_copy(c, slot).wait()
        if c < PE_CH:
            pe_copy(p).wait()
        if c >= DEPTH:
            o_copy(c - DEPTH, slot).wait()
        o_v[slot] = x_v[slot] + pe_v[p]
        o_copy(c, slot).start()
        if c + DEPTH < NCHUNK:
            x_copy(c + DEPTH, slot).start()

    for c in range(NCHUNK - DEPTH, NCHUNK):
        o_copy(c, c % DEPTH).wait()


def kernel(x, pe_table):
    batch, seq_len, d_model = x.shape
    pe = pe_table[:seq_len]
    x2 = x.reshape(batch * seq_len, d_model)
    out = pl.pallas_call(
        _body,
        in_specs=[
            pl.BlockSpec(memory_space=pltpu.HBM),
            pl.BlockSpec(memory_space=pltpu.HBM),
        ],
        out_specs=pl.BlockSpec(memory_space=pltpu.HBM),
        out_shape=jax.ShapeDtypeStruct((ROWS, d_model), x.dtype),
        scratch_shapes=[
            pltpu.VMEM((PE_CH, CHUNK_ROWS, D), jnp.float32),
            pltpu.VMEM((DEPTH, CHUNK_ROWS, D), jnp.float32),
            pltpu.VMEM((DEPTH, CHUNK_ROWS, D), jnp.float32),
            pltpu.SemaphoreType.DMA((PE_CH,)),
            pltpu.SemaphoreType.DMA((DEPTH,)),
            pltpu.SemaphoreType.DMA((DEPTH,)),
        ],
    )(x2, pe)
    return out.reshape(batch, seq_len, d_model)


# speedup vs baseline: 1.0188x; 1.0188x over previous
import jax
import jax.numpy as jnp
from jax.experimental import pallas as pl

B, S, D = 4, 2048, 1024


def _add_body(x_ref, pe_ref, out_ref):
    out_ref[...] = x_ref[...] + pe_ref[...]


def kernel(x, pe_table):
    batch, seq_len, d_model = x.shape
    pe = pe_table[:seq_len]
    x2 = x.reshape(batch * seq_len, d_model)
    out = pl.pallas_call(
        _add_body,
        grid=(batch,),
        in_specs=[
            pl.BlockSpec((seq_len, d_model), lambda b: (b, 0)),
            pl.BlockSpec((seq_len, d_model), lambda b: (0, 0)),
        ],
        out_specs=pl.BlockSpec((seq_len, d_model), lambda b: (b, 0)),
        out_shape=jax.ShapeDtypeStruct((batch * seq_len, d_model), x.dtype),
    )(x2, pe)
    return out.reshape(batch, seq_len, d_model)
